# baseline (speedup 1.0000x reference)
import functools

import jax
import jax.numpy as jnp
from jax import lax
from jax.experimental import pallas as pl
from jax.experimental.pallas import tpu as pltpu
from jax.experimental.pallas import tpu_sc as plsc

_B, _N, _M, _K, _C = 4, 16384, 1024, 32, 128
_MB = 64
_MK = _M * _K
_NCH = 3 + _C
_NW = 32
_JOBS = _B * _NCH
_JPW = (_JOBS + _NW - 1) // _NW


def _topk_body(c_ref, p_ref, idx_ref, d_ref):
    cb = c_ref[0]
    pT = p_ref[0]
    cp = jnp.dot(cb, pT, preferred_element_type=jnp.float32)
    fx, fy, fz = cb[:, 0:1], cb[:, 1:2], cb[:, 2:3]
    gx, gy, gz = pT[0:1, :], pT[1:2, :], pT[2:3, :]
    c2 = (fx * fx + fy * fy) + fz * fz
    p2 = (gx * gx + gy * gy) + gz * gz
    d_ref[...] = (c2 + p2) - 2.0 * cp
    iota = lax.broadcasted_iota(jnp.int32, (_MB, _N), 1)
    kio = lax.broadcasted_iota(jnp.int32, (_MB, _K), 1)
    inf = jnp.float32(jnp.inf)

    def step(k, carry):
        pv, pi = carry
        d = d_ref[...]
        mask = (d > pv) | ((d == pv) & (iota > pi))
        v = jnp.min(jnp.where(mask, d, inf), axis=1, keepdims=True)
        i = jnp.min(jnp.where(mask & (d == v), iota, _N), axis=1,
                    keepdims=True)
        idx_ref[0] = jnp.where(kio == k, i, idx_ref[0])
        return v, i

    lax.fori_loop(0, _K, step,
                  (jnp.full((_MB, 1), -jnp.inf, jnp.float32),
                   jnp.full((_MB, 1), -1, jnp.int32)))


def _knn_idx_tc(center_xyz, points_T):
    return pl.pallas_call(
        _topk_body,
        grid=(_B, _M // _MB),
        in_specs=[
            pl.BlockSpec((1, _MB, 3), lambda b, i: (b, i, 0)),
            pl.BlockSpec((1, 3, _N), lambda b, i: (b, 0, 0)),
        ],
        out_specs=pl.BlockSpec((1, _MB, _K), lambda b, i: (b, i, 0)),
        out_shape=jax.ShapeDtypeStruct((_B, _M, _K), jnp.int32),
        scratch_shapes=[pltpu.VMEM((_MB, _N), jnp.float32)],
    )(center_xyz, points_T)


@functools.partial(
    pl.kernel,
    out_type=jax.ShapeDtypeStruct((_B, _NCH, _MK), jnp.float32),
    mesh=plsc.VectorSubcoreMesh(core_axis_name="c", subcore_axis_name="s"),
    scratch_types=[
        pltpu.VMEM((_MK,), jnp.int32),
        pltpu.VMEM((_N,), jnp.float32),
        pltpu.VMEM((_M,), jnp.float32),
        pltpu.VMEM((_MK,), jnp.float32),
    ],
    compiler_params=pltpu.CompilerParams(needs_layout_passes=False),
)
def _gather_sc(feat_hbm, pts_hbm, ctr_hbm, idx_hbm, out_hbm,
               idx_v, col_v, ctr_v, out_v):
    cid = lax.axis_index("c")
    sid = lax.axis_index("s")
    wid = sid * 2 + cid
    iota16 = lax.iota(jnp.int32, 16)

    def do_job(t, carry):
        job = wid * _JPW + t

        @pl.when(job < _JOBS)
        def _():
            b = lax.div(job, _NCH)
            c = lax.rem(job, _NCH)
            pltpu.sync_copy(idx_hbm.at[b], idx_v)

            @pl.when(c < 3)
            def _():
                pltpu.sync_copy(pts_hbm.at[b, c], col_v)
                pltpu.sync_copy(ctr_hbm.at[b, c], ctr_v)

                def g(j, carry2):
                    iv = idx_v[pl.ds(j * 16, 16)]
                    vals = plsc.load_gather(col_v, [iv])
                    mpos = lax.shift_right_logical(j * 16 + iota16, 5)
                    cv = plsc.load_gather(ctr_v, [mpos])
                    out_v[pl.ds(j * 16, 16)] = vals - cv
                    return carry2

                lax.fori_loop(0, _MK // 16, g, 0)

            @pl.when(c >= 3)
            def _():
                pltpu.sync_copy(feat_hbm.at[b, c - 3], col_v)

                def g(j, carry2):
                    iv = idx_v[pl.ds(j * 16, 16)]
                    out_v[pl.ds(j * 16, 16)] = plsc.load_gather(col_v, [iv])
                    return carry2

                lax.fori_loop(0, _MK // 16, g, 0)

            pltpu.sync_copy(out_v, out_hbm.at[b, c])
        return carry

    lax.fori_loop(0, _JPW, do_job, 0)


def kernel(points_xyz, center_xyz, features):
    pts_T = jnp.transpose(points_xyz, (0, 2, 1))
    ctr_T = jnp.transpose(center_xyz, (0, 2, 1))
    idx = _knn_idx_tc(center_xyz, pts_T)
    out = _gather_sc(features, pts_T, ctr_T, idx.reshape(_B, _MK))
    return out.reshape(_B, _NCH, _M, _K)


# speedup vs baseline: 1.3749x; 1.3749x over previous
import functools

import jax
import jax.numpy as jnp
from jax import lax
from jax.experimental import pallas as pl
from jax.experimental.pallas import tpu as pltpu
from jax.experimental.pallas import tpu_sc as plsc

_B, _N, _M, _K, _C = 4, 16384, 1024, 32, 128
_MB = 64
_MK = _M * _K
_NCH = 3 + _C
_NW = 32
_JOBS = _B * _NCH
_JPW = (_JOBS + _NW - 1) // _NW


def _topk_body(c_ref, p_ref, idx_ref, d_ref):
    cb = c_ref[0]
    pT = p_ref[0]
    cp = jnp.dot(cb, pT, preferred_element_type=jnp.float32)
    fx, fy, fz = cb[:, 0:1], cb[:, 1:2], cb[:, 2:3]
    gx, gy, gz = pT[0:1, :], pT[1:2, :], pT[2:3, :]
    c2 = (fx * fx + fy * fy) + fz * fz
    p2 = (gx * gx + gy * gy) + gz * gz
    d_ref[...] = (c2 + p2) - 2.0 * cp
    iota = lax.broadcasted_iota(jnp.int32, (_MB, _N), 1)
    kio = lax.broadcasted_iota(jnp.int32, (_MB, _K), 1)

    def step(k, carry):
        d = d_ref[...]
        v = jnp.min(d, axis=1, keepdims=True)
        i = jnp.min(jnp.where(d == v, iota, _N), axis=1)
        idx_ref[0] = jnp.where(kio == k, i[:, None], idx_ref[0])
        d_ref[...] = jnp.where(iota == i[:, None], jnp.float32(jnp.inf), d)
        return carry

    lax.fori_loop(0, _K, step, 0)


def _knn_idx_tc(center_xyz, points_T):
    return pl.pallas_call(
        _topk_body,
        grid=(_B, _M // _MB),
        in_specs=[
            pl.BlockSpec((1, _MB, 3), lambda b, i: (b, i, 0)),
            pl.BlockSpec((1, 3, _N), lambda b, i: (b, 0, 0)),
        ],
        out_specs=pl.BlockSpec((1, _MB, _K), lambda b, i: (b, i, 0)),
        out_shape=jax.ShapeDtypeStruct((_B, _M, _K), jnp.int32),
        scratch_shapes=[pltpu.VMEM((_MB, _N), jnp.float32)],
    )(center_xyz, points_T)


@functools.partial(
    pl.kernel,
    out_type=jax.ShapeDtypeStruct((_B, _NCH, _MK), jnp.float32),
    mesh=plsc.VectorSubcoreMesh(core_axis_name="c", subcore_axis_name="s"),
    scratch_types=[
        pltpu.VMEM((_MK,), jnp.int32),
        pltpu.VMEM((_N,), jnp.float32),
        pltpu.VMEM((_M,), jnp.float32),
        pltpu.VMEM((_MK,), jnp.float32),
    ],
    compiler_params=pltpu.CompilerParams(needs_layout_passes=False),
)
def _gather_sc(feat_hbm, pts_hbm, ctr_hbm, idx_hbm, out_hbm,
               idx_v, col_v, ctr_v, out_v):
    cid = lax.axis_index("c")
    sid = lax.axis_index("s")
    wid = sid * 2 + cid
    iota16 = lax.iota(jnp.int32, 16)

    def do_job(t, carry):
        job = wid * _JPW + t

        @pl.when(job < _JOBS)
        def _():
            b = lax.div(job, _NCH)
            c = lax.rem(job, _NCH)
            pltpu.sync_copy(idx_hbm.at[b], idx_v)

            @pl.when(c < 3)
            def _():
                pltpu.sync_copy(pts_hbm.at[b, c], col_v)
                pltpu.sync_copy(ctr_hbm.at[b, c], ctr_v)

                def g(j, carry2):
                    iv = idx_v[pl.ds(j * 16, 16)]
                    vals = plsc.load_gather(col_v, [iv])
                    mpos = lax.shift_right_logical(j * 16 + iota16, 5)
                    cv = plsc.load_gather(ctr_v, [mpos])
                    out_v[pl.ds(j * 16, 16)] = vals - cv
                    return carry2

                lax.fori_loop(0, _MK // 16, g, 0)

            @pl.when(c >= 3)
            def _():
                pltpu.sync_copy(feat_hbm.at[b, c - 3], col_v)

                def g(j, carry2):
                    iv = idx_v[pl.ds(j * 16, 16)]
                    out_v[pl.ds(j * 16, 16)] = plsc.load_gather(col_v, [iv])
                    return carry2

                lax.fori_loop(0, _MK // 16, g, 0)

            pltpu.sync_copy(out_v, out_hbm.at[b, c])
        return carry

    lax.fori_loop(0, _JPW, do_job, 0)


def kernel(points_xyz, center_xyz, features):
    pts_T = jnp.transpose(points_xyz, (0, 2, 1))
    ctr_T = jnp.transpose(center_xyz, (0, 2, 1))
    idx = _knn_idx_tc(center_xyz, pts_T)
    out = _gather_sc(features, pts_T, ctr_T, idx.reshape(_B, _MK))
    return out.reshape(_B, _NCH, _M, _K)
